# baseline (speedup 1.0000x reference)
import functools

import jax
import jax.numpy as jnp
from jax import lax
from jax.experimental import pallas as pl
from jax.experimental.pallas import tpu as pltpu
from jax.experimental.pallas import tpu_sc as plsc

N = 10000
M = 12
A = 128
Bf = 64
K = 3
D = 2 * A + Bf
KD = K * D
KA = K * A
NCRYS = 64

TA = 200
GRID = N // TA
TE = TA * M

_NW = 32
_GB = 128
_CHUNKS = 30
_MC = 3
_NMAC = _CHUNKS // _MC
_MROWS = _MC * _GB
_BPAD = _NW * _GB * _CHUNKS


def _sc_gather(x, idx_pad):
    mesh = plsc.VectorSubcoreMesh(core_axis_name="c", subcore_axis_name="s")
    idx_r = idx_pad.reshape(_NW, _CHUNKS, _GB)

    @functools.partial(
        pl.kernel,
        mesh=mesh,
        out_type=jax.ShapeDtypeStruct((_BPAD, A), jnp.float32),
        scratch_types=[
            pltpu.VMEM((_CHUNKS, _GB), jnp.int32),
            pltpu.VMEM((2, _MROWS, A), jnp.float32),
            pltpu.SemaphoreType.DMA,
            pltpu.SemaphoreType.DMA,
            pltpu.SemaphoreType.DMA,
            pltpu.SemaphoreType.DMA,
        ],
    )
    def k(x_hbm, idx_hbm, out_hbm, idx_all, rows_v, gsem0, gsem1,
          wsem0, wsem1):
        wid = lax.axis_index("s") * 2 + lax.axis_index("c")
        base = wid * (_GB * _CHUNKS)
        pltpu.sync_copy(idx_hbm.at[wid], idx_all)
        gsems = (gsem0, gsem1)
        wsems = (wsem0, wsem1)

        def pair(p, carry):
            @pl.when(p > 0)
            def _():
                for b in range(2):
                    pltpu.make_async_copy(
                        rows_v.at[b],
                        out_hbm.at[pl.ds(0, _MROWS)],
                        wsems[b],
                    ).wait()

            handles = []
            for b in range(2):
                m = 2 * p + b
                hs = []
                for j in range(_MC):
                    hs.append(pltpu.async_copy(
                        x_hbm.at[idx_all.at[m * _MC + j]],
                        rows_v.at[b, pl.ds(j * _GB, _GB)],
                        gsems[b],
                    ))
                handles.append(hs)
            for b in range(2):
                m = 2 * p + b
                for h in handles[b]:
                    h.wait()
                pltpu.async_copy(
                    rows_v.at[b],
                    out_hbm.at[pl.ds(base + m * _MROWS, _MROWS)],
                    wsems[b],
                )
            return carry

        lax.fori_loop(0, _NMAC // 2, pair, 0)
        for b in range(2):
            pltpu.make_async_copy(
                rows_v.at[b],
                out_hbm.at[pl.ds(0, _MROWS)],
                wsems[b],
            ).wait()

    return k(x, idx_r)


def _prep_body(af_ref, nf_ref, w_ref, b_ref, x_ref, bond_ref):
    x_ref[...] = (
        jnp.dot(af_ref[...], w_ref[...], preferred_element_type=jnp.float32)
        + b_ref[...]
    )
    nf = nf_ref[...]
    nrm = jnp.sqrt(jnp.sum(nf * nf, axis=1, keepdims=True))
    bond_ref[...] = nf / jnp.maximum(nrm, 1e-12)


def _prep(atom_fea, nbr_fea, emb_W, emb_b):
    orig = atom_fea.shape[1]
    return pl.pallas_call(
        _prep_body,
        grid=(GRID,),
        in_specs=[
            pl.BlockSpec((TA, orig), lambda i: (i, 0)),
            pl.BlockSpec((TA, M, Bf), lambda i: (i, 0, 0)),
            pl.BlockSpec((orig, A), lambda i: (0, 0)),
            pl.BlockSpec((1, A), lambda i: (0, 0)),
        ],
        out_specs=[
            pl.BlockSpec((TA, A), lambda i: (i, 0)),
            pl.BlockSpec((TA, M, Bf), lambda i: (i, 0, 0)),
        ],
        out_shape=[
            jax.ShapeDtypeStruct((N, A), jnp.float32),
            jax.ShapeDtypeStruct((N, M, Bf), jnp.float32),
        ],
        compiler_params=pltpu.CompilerParams(
            dimension_semantics=("arbitrary",)
        ),
    )(atom_fea, nbr_fea, emb_W, emb_b)


def _compute_g(x_t, gath_t, bond_t, w_all, bias):
    self_rep = jnp.broadcast_to(x_t[:, None, :], (TA, M, A)).reshape(TE, A)
    total = jnp.concatenate(
        [self_rep, gath_t, bond_t.reshape(TE, Bf)], axis=1)
    g = jnp.dot(total, w_all, preferred_element_type=jnp.float32)
    g = g.reshape(TA, M, KD) + bias.reshape(1, 1, KD)
    return g


def _kahan_add(acc, comp, s):
    y = s - comp[...]
    t = acc[...] + y
    comp[...] = (t - acc[...]) - y
    acc[...] = t


def _stats_body(x_ref, gath_ref, bond_ref, w_ref, b_ref,
                c_ref, ssum_ref, ssq_ref, c_s, acc_s, acc_q, cmp_s, cmp_q):
    g = _compute_g(x_ref[...], gath_ref[...], bond_ref[...],
                   w_ref[...], b_ref[...])
    g2 = g.reshape(TE, KD)
    i = pl.program_id(0)

    @pl.when(i == 0)
    def _():
        c = jnp.sum(g2, axis=0, keepdims=True) / float(TE)
        c_s[...] = c
        d = g2 - c
        acc_s[...] = jnp.sum(d, axis=0, keepdims=True)
        acc_q[...] = jnp.sum(d * d, axis=0, keepdims=True)
        cmp_s[...] = jnp.zeros_like(c)
        cmp_q[...] = jnp.zeros_like(c)

    @pl.when(i > 0)
    def _():
        d = g2 - c_s[...]
        _kahan_add(acc_s, cmp_s, jnp.sum(d, axis=0, keepdims=True))
        _kahan_add(acc_q, cmp_q, jnp.sum(d * d, axis=0, keepdims=True))

    @pl.when(i == GRID - 1)
    def _():
        c_ref[...] = c_s[...]
        ssum_ref[...] = acc_s[...]
        ssq_ref[...] = acc_q[...]


def _stats(x, gath, bond, w_all, bias):
    return pl.pallas_call(
        _stats_body,
        grid=(GRID,),
        in_specs=[
            pl.BlockSpec((TA, A), lambda i: (i, 0)),
            pl.BlockSpec((TE, A), lambda i: (i, 0)),
            pl.BlockSpec((TA, M, Bf), lambda i: (i, 0, 0)),
            pl.BlockSpec((D, KD), lambda i: (0, 0)),
            pl.BlockSpec((1, KD), lambda i: (0, 0)),
        ],
        out_specs=[
            pl.BlockSpec((1, KD), lambda i: (0, 0)),
            pl.BlockSpec((1, KD), lambda i: (0, 0)),
            pl.BlockSpec((1, KD), lambda i: (0, 0)),
        ],
        out_shape=[
            jax.ShapeDtypeStruct((1, KD), jnp.float32),
            jax.ShapeDtypeStruct((1, KD), jnp.float32),
            jax.ShapeDtypeStruct((1, KD), jnp.float32),
        ],
        scratch_shapes=[
            pltpu.VMEM((1, KD), jnp.float32),
            pltpu.VMEM((1, KD), jnp.float32),
            pltpu.VMEM((1, KD), jnp.float32),
            pltpu.VMEM((1, KD), jnp.float32),
            pltpu.VMEM((1, KD), jnp.float32),
        ],
        compiler_params=pltpu.CompilerParams(
            dimension_semantics=("arbitrary",)
        ),
    )(x, gath, bond, w_all, bias)


def _main_body(x_ref, gath_ref, bond_ref, w_ref, b_ref,
               c_ref, ssum_ref, ssq_ref, g1g_ref, g1b_ref, wfc_ref, bfc_ref,
               pre_ref, bondn_ref, c2_ref, s2sum_ref, s2sq_ref,
               c2_s, acc_s, acc_q, cmp_s, cmp_q):
    cnt = float(N * M)
    mu_c = ssum_ref[...] / cnt
    mean = c_ref[...] + mu_c
    var = ssq_ref[...] / cnt - mu_c * mu_c
    sd = jnp.sqrt(var + 1e-5)

    g = _compute_g(x_ref[...], gath_ref[...], bond_ref[...],
                   w_ref[...], b_ref[...])
    g = ((g - mean.reshape(1, 1, KD)) / sd.reshape(1, 1, KD)
         * g1g_ref[...].reshape(1, 1, KD) + g1b_ref[...].reshape(1, 1, KD))

    bond = bond_ref[...]
    pres = []
    nns = []
    for i in range(K):
        f = g[:, :, i * D:i * D + A]
        c = g[:, :, i * D + A:i * D + 2 * A]
        nn = g[:, :, i * D + 2 * A:(i + 1) * D]
        m = jnp.max(f, axis=1, keepdims=True)
        e = jnp.exp(f - m)
        filt = e / jnp.sum(e, axis=1, keepdims=True)
        pres.append(jnp.sum(filt * jnp.maximum(c, 0.0), axis=1))
        nns.append(nn + bond)
    pre = jnp.concatenate(pres, axis=1)
    pre_ref[...] = pre

    i0 = pl.program_id(0)

    @pl.when(i0 == 0)
    def _():
        c2 = jnp.sum(pre, axis=0, keepdims=True) / float(TA)
        c2_s[...] = c2
        d = pre - c2
        acc_s[...] = jnp.sum(d, axis=0, keepdims=True)
        acc_q[...] = jnp.sum(d * d, axis=0, keepdims=True)
        cmp_s[...] = jnp.zeros_like(c2)
        cmp_q[...] = jnp.zeros_like(c2)

    @pl.when(i0 > 0)
    def _():
        d = pre - c2_s[...]
        _kahan_add(acc_s, cmp_s, jnp.sum(d, axis=0, keepdims=True))
        _kahan_add(acc_q, cmp_q, jnp.sum(d * d, axis=0, keepdims=True))

    @pl.when(i0 == GRID - 1)
    def _():
        c2_ref[...] = c2_s[...]
        s2sum_ref[...] = acc_s[...]
        s2sq_ref[...] = acc_q[...]

    nns_bf = [v.astype(jnp.bfloat16).astype(jnp.float32) for v in nns]
    gated = []
    for j in range(2 * K):
        gj = None
        for i in range(K):
            w = jnp.full((1, 1, 1), wfc_ref[i, j], jnp.float32).astype(
                jnp.bfloat16).astype(jnp.float32)
            gj = nns_bf[i] * w if gj is None else gj + nns_bf[i] * w
        gated.append(gj + bfc_ref[j])
    mg = jnp.maximum(jnp.maximum(gated[K], gated[K + 1]), gated[K + 2])
    es = [jnp.exp(gated[K + t] - mg) for t in range(K)]
    se = es[0] + es[1] + es[2]
    bondn_ref[...] = (gated[0] * es[0] + gated[1] * es[1]
                      + gated[2] * es[2]) / se


def _main(x, gath, bond, w_all, bias, c, ssum, ssq, g1g, g1b, wfc, bfc):
    return pl.pallas_call(
        _main_body,
        grid=(GRID,),
        in_specs=[
            pl.BlockSpec((TA, A), lambda i: (i, 0)),
            pl.BlockSpec((TE, A), lambda i: (i, 0)),
            pl.BlockSpec((TA, M, Bf), lambda i: (i, 0, 0)),
            pl.BlockSpec((D, KD), lambda i: (0, 0)),
            pl.BlockSpec((1, KD), lambda i: (0, 0)),
            pl.BlockSpec((1, KD), lambda i: (0, 0)),
            pl.BlockSpec((1, KD), lambda i: (0, 0)),
            pl.BlockSpec((1, KD), lambda i: (0, 0)),
            pl.BlockSpec((1, KD), lambda i: (0, 0)),
            pl.BlockSpec((1, KD), lambda i: (0, 0)),
            pl.BlockSpec(memory_space=pltpu.SMEM),
            pl.BlockSpec(memory_space=pltpu.SMEM),
        ],
        out_specs=[
            pl.BlockSpec((TA, KA), lambda i: (i, 0)),
            pl.BlockSpec((TA, M, Bf), lambda i: (i, 0, 0)),
            pl.BlockSpec((1, KA), lambda i: (0, 0)),
            pl.BlockSpec((1, KA), lambda i: (0, 0)),
            pl.BlockSpec((1, KA), lambda i: (0, 0)),
        ],
        out_shape=[
            jax.ShapeDtypeStruct((N, KA), jnp.float32),
            jax.ShapeDtypeStruct((N, M, Bf), jnp.float32),
            jax.ShapeDtypeStruct((1, KA), jnp.float32),
            jax.ShapeDtypeStruct((1, KA), jnp.float32),
            jax.ShapeDtypeStruct((1, KA), jnp.float32),
        ],
        scratch_shapes=[
            pltpu.VMEM((1, KA), jnp.float32),
            pltpu.VMEM((1, KA), jnp.float32),
            pltpu.VMEM((1, KA), jnp.float32),
            pltpu.VMEM((1, KA), jnp.float32),
            pltpu.VMEM((1, KA), jnp.float32),
        ],
        compiler_params=pltpu.CompilerParams(
            dimension_semantics=("arbitrary",)
        ),
    )(x, gath, bond, w_all, bias, c, ssum, ssq, g1g, g1b, wfc, bfc)


def _atom_body(x_ref, pre_ref, c2_ref, s2sum_ref, s2sq_ref, g2g_ref, g2b_ref,
               wfc_ref, bfc_ref, xn_ref):
    cnt = float(N)
    mu_c = s2sum_ref[...] / cnt
    mean = c2_ref[...] + mu_c
    var = s2sq_ref[...] / cnt - mu_c * mu_c
    pre = ((pre_ref[...] - mean) / jnp.sqrt(var + 1e-5)
           * g2g_ref[...] + g2b_ref[...])
    x = x_ref[...]
    outs = [x + pre[:, i * A:(i + 1) * A] for i in range(K)]
    outs_bf = [v.astype(jnp.bfloat16).astype(jnp.float32) for v in outs]
    gated = []
    for j in range(2 * K):
        gj = None
        for i in range(K):
            w = jnp.full((1, 1), wfc_ref[i, j], jnp.float32).astype(
                jnp.bfloat16).astype(jnp.float32)
            gj = outs_bf[i] * w if gj is None else gj + outs_bf[i] * w
        gated.append(gj + bfc_ref[j])
    mg = jnp.maximum(jnp.maximum(gated[K], gated[K + 1]), gated[K + 2])
    es = [jnp.exp(gated[K + t] - mg) for t in range(K)]
    se = es[0] + es[1] + es[2]
    xn_ref[...] = (gated[0] * es[0] + gated[1] * es[1]
                   + gated[2] * es[2]) / se


def _atom(x, pre, c2, s2sum, s2sq, g2g, g2b, wfc, bfc):
    return pl.pallas_call(
        _atom_body,
        grid=(GRID,),
        in_specs=[
            pl.BlockSpec((TA, A), lambda i: (i, 0)),
            pl.BlockSpec((TA, KA), lambda i: (i, 0)),
            pl.BlockSpec((1, KA), lambda i: (0, 0)),
            pl.BlockSpec((1, KA), lambda i: (0, 0)),
            pl.BlockSpec((1, KA), lambda i: (0, 0)),
            pl.BlockSpec((1, KA), lambda i: (0, 0)),
            pl.BlockSpec((1, KA), lambda i: (0, 0)),
            pl.BlockSpec(memory_space=pltpu.SMEM),
            pl.BlockSpec(memory_space=pltpu.SMEM),
        ],
        out_specs=pl.BlockSpec((TA, A), lambda i: (i, 0)),
        out_shape=jax.ShapeDtypeStruct((N, A), jnp.float32),
        compiler_params=pltpu.CompilerParams(
            dimension_semantics=("arbitrary",)
        ),
    )(x, pre, c2, s2sum, s2sq, g2g, g2b, wfc, bfc)


def _pool_body(x_ref, oh_ref, w1_ref, b1_ref, w2_ref, b2_ref, w3_ref, b3_ref,
               wo_ref, bo_ref, out_ref, acc, cacc):
    oh = oh_ref[...]
    x = x_ref[...]
    dn = (((0,), (0,)), ((), ()))
    ps = lax.dot_general(oh, x, dn, preferred_element_type=jnp.float32,
                         precision=lax.Precision.HIGHEST)
    pc = lax.dot_general(oh, jnp.ones_like(x), dn,
                         preferred_element_type=jnp.float32,
                         precision=lax.Precision.HIGHEST)
    i = pl.program_id(0)

    @pl.when(i == 0)
    def _():
        acc[...] = ps
        cacc[...] = pc

    @pl.when(i > 0)
    def _():
        acc[...] = acc[...] + ps
        cacc[...] = cacc[...] + pc

    @pl.when(i == GRID - 1)
    def _():
        crys = acc[...] / jnp.maximum(cacc[...], 1.0)
        crys = jnp.maximum(crys, 0.0)
        crys = jnp.dot(crys, w1_ref[...],
                       preferred_element_type=jnp.float32) + b1_ref[...]
        crys = jnp.maximum(crys, 0.0)
        crys = jnp.dot(crys, w2_ref[...],
                       preferred_element_type=jnp.float32) + b2_ref[...]
        crys = jnp.maximum(crys, 0.0)
        crys = jnp.dot(crys, w3_ref[...],
                       preferred_element_type=jnp.float32) + b3_ref[...]
        crys = jnp.maximum(crys, 0.0)
        crys_bf = crys.astype(jnp.bfloat16).astype(jnp.float32)
        wo_bf = wo_ref[...].astype(jnp.bfloat16).astype(jnp.float32)
        out_ref[...] = jnp.sum(crys_bf * wo_bf, axis=1,
                               keepdims=True) + bo_ref[0]


def _pool(x, onehot, w1, b1, w2, b2, w3, b3, woT, bo):
    dim2 = w1.shape[1]
    return pl.pallas_call(
        _pool_body,
        grid=(GRID,),
        in_specs=[
            pl.BlockSpec((TA, A), lambda i: (i, 0)),
            pl.BlockSpec((TA, NCRYS), lambda i: (i, 0)),
            pl.BlockSpec((A, dim2), lambda i: (0, 0)),
            pl.BlockSpec((1, dim2), lambda i: (0, 0)),
            pl.BlockSpec((dim2, dim2), lambda i: (0, 0)),
            pl.BlockSpec((1, dim2), lambda i: (0, 0)),
            pl.BlockSpec((dim2, dim2), lambda i: (0, 0)),
            pl.BlockSpec((1, dim2), lambda i: (0, 0)),
            pl.BlockSpec((1, dim2), lambda i: (0, 0)),
            pl.BlockSpec(memory_space=pltpu.SMEM),
        ],
        out_specs=pl.BlockSpec((NCRYS, 1), lambda i: (0, 0)),
        out_shape=jax.ShapeDtypeStruct((NCRYS, 1), jnp.float32),
        scratch_shapes=[
            pltpu.VMEM((NCRYS, A), jnp.float32),
            pltpu.VMEM((NCRYS, A), jnp.float32),
        ],
        compiler_params=pltpu.CompilerParams(
            dimension_semantics=("arbitrary",)
        ),
    )(x, onehot, w1, b1, w2, b2, w3, b3, woT, bo)


def _forward_impl(atom_fea, nbr_fea, nbr_fea_idx, batch, params, gather_fn):
    x, bond = _prep(atom_fea, nbr_fea, params["emb_W"],
                    params["emb_b"].reshape(1, A))
    idx_flat = nbr_fea_idx.reshape(-1).astype(jnp.int32)
    idx_pad = jnp.concatenate(
        [idx_flat, jnp.zeros((_BPAD - N * M,), jnp.int32)])
    onehot = (batch[:, None] == jnp.arange(NCRYS, dtype=batch.dtype)[None, :]
              ).astype(jnp.float32)

    for p in params["convs"]:
        w_all = jnp.concatenate([p["fc_W"][i] for i in range(K)], axis=1)
        bias = jnp.concatenate(p["fc_b"]).reshape(1, KD)
        g1g = jnp.concatenate(p["bn1_g"]).reshape(1, KD)
        g1b = jnp.concatenate(p["bn1_b"]).reshape(1, KD)
        g2g = jnp.concatenate(p["bn2_g"]).reshape(1, KA)
        g2b = jnp.concatenate(p["bn2_b"]).reshape(1, KA)

        gath = gather_fn(x, idx_pad)[:N * M]
        c, ssum, ssq = _stats(x, gath, bond, w_all, bias)
        pre, bond, c2, s2sum, s2sq = _main(x, gath, bond, w_all, bias,
                                           c, ssum, ssq, g1g, g1b,
                                           p["nbr_fc_W"], p["nbr_fc_b"])
        x = _atom(x, pre, c2, s2sum, s2sq, g2g, g2b,
                  p["atom_fc_W"], p["atom_fc_b"])

    return _pool(x, onehot,
                 params["c2f_W"], params["c2f_b"].reshape(1, -1),
                 params["fcs_W"][0], params["fcs_b"][0].reshape(1, -1),
                 params["fcs_W"][1], params["fcs_b"][1].reshape(1, -1),
                 params["out_W"].reshape(1, -1), params["out_b"])


@jax.jit
def kernel(atom_fea, nbr_fea, nbr_fea_idx, batch, params):
    return _forward_impl(atom_fea, nbr_fea, nbr_fea_idx, batch, params,
                         _sc_gather)


# speedup vs baseline: 1.0159x; 1.0159x over previous
import functools

import jax
import jax.numpy as jnp
from jax import lax
from jax.experimental import pallas as pl
from jax.experimental.pallas import tpu as pltpu
from jax.experimental.pallas import tpu_sc as plsc

N = 10000
M = 12
A = 128
Bf = 64
K = 3
D = 2 * A + Bf
KD = K * D
KA = K * A
NCRYS = 64

TA = 200
GRID = N // TA
TE = TA * M

_NW = 32
_GB = 128
_CHUNKS = 30
_MC = 3
_NMAC = _CHUNKS // _MC
_MROWS = _MC * _GB
_BPAD = _NW * _GB * _CHUNKS


def _sc_gather(x, idx_pad):
    mesh = plsc.VectorSubcoreMesh(core_axis_name="c", subcore_axis_name="s")
    idx_r = idx_pad.reshape(_NW, _CHUNKS, _GB)

    @functools.partial(
        pl.kernel,
        mesh=mesh,
        out_type=jax.ShapeDtypeStruct((_BPAD, A), jnp.float32),
        scratch_types=[
            pltpu.VMEM((_CHUNKS, _GB), jnp.int32),
            pltpu.VMEM((2, _MROWS, A), jnp.float32),
            pltpu.SemaphoreType.DMA,
            pltpu.SemaphoreType.DMA,
            pltpu.SemaphoreType.DMA,
            pltpu.SemaphoreType.DMA,
        ],
    )
    def k(x_hbm, idx_hbm, out_hbm, idx_all, rows_v, gsem0, gsem1,
          wsem0, wsem1):
        wid = lax.axis_index("s") * 2 + lax.axis_index("c")
        base = wid * (_GB * _CHUNKS)
        pltpu.sync_copy(idx_hbm.at[wid], idx_all)
        gsems = (gsem0, gsem1)
        wsems = (wsem0, wsem1)

        def pair(p, carry):
            @pl.when(p > 0)
            def _():
                for b in range(2):
                    pltpu.make_async_copy(
                        rows_v.at[b],
                        out_hbm.at[pl.ds(0, _MROWS)],
                        wsems[b],
                    ).wait()

            handles = []
            for b in range(2):
                m = 2 * p + b
                hs = []
                for j in range(_MC):
                    hs.append(pltpu.async_copy(
                        x_hbm.at[idx_all.at[m * _MC + j]],
                        rows_v.at[b, pl.ds(j * _GB, _GB)],
                        gsems[b],
                    ))
                handles.append(hs)
            for b in range(2):
                m = 2 * p + b
                for h in handles[b]:
                    h.wait()
                pltpu.async_copy(
                    rows_v.at[b],
                    out_hbm.at[pl.ds(base + m * _MROWS, _MROWS)],
                    wsems[b],
                )
            return carry

        lax.fori_loop(0, _NMAC // 2, pair, 0)
        for b in range(2):
            pltpu.make_async_copy(
                rows_v.at[b],
                out_hbm.at[pl.ds(0, _MROWS)],
                wsems[b],
            ).wait()

    return k(x, idx_r)


def _prep_body(af_ref, nf_ref, w_ref, b_ref, x_ref, bond_ref):
    x_ref[...] = (
        jnp.dot(af_ref[...], w_ref[...], preferred_element_type=jnp.float32)
        + b_ref[...]
    )
    nf = nf_ref[...]
    nrm = jnp.sqrt(jnp.sum(nf * nf, axis=1, keepdims=True))
    bond_ref[...] = nf / jnp.maximum(nrm, 1e-12)


def _prep(atom_fea, nbr_fea, emb_W, emb_b):
    orig = atom_fea.shape[1]
    return pl.pallas_call(
        _prep_body,
        grid=(GRID,),
        in_specs=[
            pl.BlockSpec((TA, orig), lambda i: (i, 0)),
            pl.BlockSpec((TA, M, Bf), lambda i: (i, 0, 0)),
            pl.BlockSpec((orig, A), lambda i: (0, 0)),
            pl.BlockSpec((1, A), lambda i: (0, 0)),
        ],
        out_specs=[
            pl.BlockSpec((TA, A), lambda i: (i, 0)),
            pl.BlockSpec((TA, M, Bf), lambda i: (i, 0, 0)),
        ],
        out_shape=[
            jax.ShapeDtypeStruct((N, A), jnp.float32),
            jax.ShapeDtypeStruct((N, M, Bf), jnp.float32),
        ],
        compiler_params=pltpu.CompilerParams(
            dimension_semantics=("arbitrary",)
        ),
    )(atom_fea, nbr_fea, emb_W, emb_b)


def _compute_g(x_t, gath_t, bond_t, w_all, bias):
    self_rep = jnp.broadcast_to(x_t[:, None, :], (TA, M, A)).reshape(TE, A)
    total = jnp.concatenate(
        [self_rep, gath_t, bond_t.reshape(TE, Bf)], axis=1)
    g = jnp.dot(total, w_all, preferred_element_type=jnp.float32)
    g = g.reshape(TA, M, KD) + bias.reshape(1, 1, KD)
    return g


def _kahan_add(acc, comp, s):
    y = s - comp[...]
    t = acc[...] + y
    comp[...] = (t - acc[...]) - y
    acc[...] = t


def _stats_body(x_ref, gath_ref, bond_ref, w_ref, b_ref,
                c_ref, ssum_ref, ssq_ref, g_ref,
                c_s, acc_s, acc_q, cmp_s, cmp_q):
    g = _compute_g(x_ref[...], gath_ref[...], bond_ref[...],
                   w_ref[...], b_ref[...])
    g2 = g.reshape(TE, KD)
    g_ref[...] = g2
    i = pl.program_id(0)

    @pl.when(i == 0)
    def _():
        c = jnp.sum(g2, axis=0, keepdims=True) / float(TE)
        c_s[...] = c
        d = g2 - c
        acc_s[...] = jnp.sum(d, axis=0, keepdims=True)
        acc_q[...] = jnp.sum(d * d, axis=0, keepdims=True)
        cmp_s[...] = jnp.zeros_like(c)
        cmp_q[...] = jnp.zeros_like(c)

    @pl.when(i > 0)
    def _():
        d = g2 - c_s[...]
        _kahan_add(acc_s, cmp_s, jnp.sum(d, axis=0, keepdims=True))
        _kahan_add(acc_q, cmp_q, jnp.sum(d * d, axis=0, keepdims=True))

    @pl.when(i == GRID - 1)
    def _():
        c_ref[...] = c_s[...]
        ssum_ref[...] = acc_s[...]
        ssq_ref[...] = acc_q[...]


def _stats(x, gath, bond, w_all, bias):
    return pl.pallas_call(
        _stats_body,
        grid=(GRID,),
        in_specs=[
            pl.BlockSpec((TA, A), lambda i: (i, 0)),
            pl.BlockSpec((TE, A), lambda i: (i, 0)),
            pl.BlockSpec((TA, M, Bf), lambda i: (i, 0, 0)),
            pl.BlockSpec((D, KD), lambda i: (0, 0)),
            pl.BlockSpec((1, KD), lambda i: (0, 0)),
        ],
        out_specs=[
            pl.BlockSpec((1, KD), lambda i: (0, 0)),
            pl.BlockSpec((1, KD), lambda i: (0, 0)),
            pl.BlockSpec((1, KD), lambda i: (0, 0)),
            pl.BlockSpec((TE, KD), lambda i: (i, 0)),
        ],
        out_shape=[
            jax.ShapeDtypeStruct((1, KD), jnp.float32),
            jax.ShapeDtypeStruct((1, KD), jnp.float32),
            jax.ShapeDtypeStruct((1, KD), jnp.float32),
            jax.ShapeDtypeStruct((N * M, KD), jnp.float32),
        ],
        scratch_shapes=[
            pltpu.VMEM((1, KD), jnp.float32),
            pltpu.VMEM((1, KD), jnp.float32),
            pltpu.VMEM((1, KD), jnp.float32),
            pltpu.VMEM((1, KD), jnp.float32),
            pltpu.VMEM((1, KD), jnp.float32),
        ],
        compiler_params=pltpu.CompilerParams(
            dimension_semantics=("arbitrary",)
        ),
    )(x, gath, bond, w_all, bias)


def _main_body(g_ref, bond_ref,
               c_ref, ssum_ref, ssq_ref, g1g_ref, g1b_ref, wg_ref, bg_ref,
               pre_ref, bondn_ref, c2_ref, s2sum_ref, s2sq_ref,
               c2_s, acc_s, acc_q, cmp_s, cmp_q):
    cnt = float(N * M)
    mu_c = ssum_ref[...] / cnt
    mean = c_ref[...] + mu_c
    var = ssq_ref[...] / cnt - mu_c * mu_c
    sd = jnp.sqrt(var + 1e-5)

    g = g_ref[...].reshape(TA, M, KD)
    g = ((g - mean.reshape(1, 1, KD)) / sd.reshape(1, 1, KD)
         * g1g_ref[...].reshape(1, 1, KD) + g1b_ref[...].reshape(1, 1, KD))

    bond = bond_ref[...]
    pres = []
    nns = []
    for i in range(K):
        f = g[:, :, i * D:i * D + A]
        c = g[:, :, i * D + A:i * D + 2 * A]
        nn = g[:, :, i * D + 2 * A:(i + 1) * D]
        m = jnp.max(f, axis=1, keepdims=True)
        e = jnp.exp(f - m)
        filt = e / jnp.sum(e, axis=1, keepdims=True)
        pres.append(jnp.sum(filt * jnp.maximum(c, 0.0), axis=1))
        nns.append(nn + bond)
    pre = jnp.concatenate(pres, axis=1)
    pre_ref[...] = pre

    i0 = pl.program_id(0)

    @pl.when(i0 == 0)
    def _():
        c2 = jnp.sum(pre, axis=0, keepdims=True) / float(TA)
        c2_s[...] = c2
        d = pre - c2
        acc_s[...] = jnp.sum(d, axis=0, keepdims=True)
        acc_q[...] = jnp.sum(d * d, axis=0, keepdims=True)
        cmp_s[...] = jnp.zeros_like(c2)
        cmp_q[...] = jnp.zeros_like(c2)

    @pl.when(i0 > 0)
    def _():
        d = pre - c2_s[...]
        _kahan_add(acc_s, cmp_s, jnp.sum(d, axis=0, keepdims=True))
        _kahan_add(acc_q, cmp_q, jnp.sum(d * d, axis=0, keepdims=True))

    @pl.when(i0 == GRID - 1)
    def _():
        c2_ref[...] = c2_s[...]
        s2sum_ref[...] = acc_s[...]
        s2sq_ref[...] = acc_q[...]

    s_bf = jnp.concatenate(nns, axis=2).reshape(TE, K * Bf).astype(
        jnp.bfloat16)
    gated = jnp.dot(s_bf, wg_ref[...].astype(jnp.bfloat16),
                    preferred_element_type=jnp.float32)
    gated = (gated.reshape(TA, M, 2 * K * Bf)
             + bg_ref[...].reshape(1, 1, 2 * K * Bf))
    core = [gated[:, :, t * Bf:(t + 1) * Bf] for t in range(K)]
    fl = [gated[:, :, (K + t) * Bf:(K + t + 1) * Bf] for t in range(K)]
    mg = jnp.maximum(jnp.maximum(fl[0], fl[1]), fl[2])
    es = [jnp.exp(fl[t] - mg) for t in range(K)]
    se = es[0] + es[1] + es[2]
    bondn_ref[...] = (core[0] * es[0] + core[1] * es[1]
                      + core[2] * es[2]) / se


def _main(g, bond, c, ssum, ssq, g1g, g1b, wg, bg):
    return pl.pallas_call(
        _main_body,
        grid=(GRID,),
        in_specs=[
            pl.BlockSpec((TE, KD), lambda i: (i, 0)),
            pl.BlockSpec((TA, M, Bf), lambda i: (i, 0, 0)),
            pl.BlockSpec((1, KD), lambda i: (0, 0)),
            pl.BlockSpec((1, KD), lambda i: (0, 0)),
            pl.BlockSpec((1, KD), lambda i: (0, 0)),
            pl.BlockSpec((1, KD), lambda i: (0, 0)),
            pl.BlockSpec((1, KD), lambda i: (0, 0)),
            pl.BlockSpec((K * Bf, 2 * K * Bf), lambda i: (0, 0)),
            pl.BlockSpec((1, 2 * K * Bf), lambda i: (0, 0)),
        ],
        out_specs=[
            pl.BlockSpec((TA, KA), lambda i: (i, 0)),
            pl.BlockSpec((TA, M, Bf), lambda i: (i, 0, 0)),
            pl.BlockSpec((1, KA), lambda i: (0, 0)),
            pl.BlockSpec((1, KA), lambda i: (0, 0)),
            pl.BlockSpec((1, KA), lambda i: (0, 0)),
        ],
        out_shape=[
            jax.ShapeDtypeStruct((N, KA), jnp.float32),
            jax.ShapeDtypeStruct((N, M, Bf), jnp.float32),
            jax.ShapeDtypeStruct((1, KA), jnp.float32),
            jax.ShapeDtypeStruct((1, KA), jnp.float32),
            jax.ShapeDtypeStruct((1, KA), jnp.float32),
        ],
        scratch_shapes=[
            pltpu.VMEM((1, KA), jnp.float32),
            pltpu.VMEM((1, KA), jnp.float32),
            pltpu.VMEM((1, KA), jnp.float32),
            pltpu.VMEM((1, KA), jnp.float32),
            pltpu.VMEM((1, KA), jnp.float32),
        ],
        compiler_params=pltpu.CompilerParams(
            dimension_semantics=("arbitrary",)
        ),
    )(g, bond, c, ssum, ssq, g1g, g1b, wg, bg)


def _atom_body(x_ref, pre_ref, c2_ref, s2sum_ref, s2sq_ref, g2g_ref, g2b_ref,
               wfc_ref, bfc_ref, xn_ref):
    cnt = float(N)
    mu_c = s2sum_ref[...] / cnt
    mean = c2_ref[...] + mu_c
    var = s2sq_ref[...] / cnt - mu_c * mu_c
    pre = ((pre_ref[...] - mean) / jnp.sqrt(var + 1e-5)
           * g2g_ref[...] + g2b_ref[...])
    x = x_ref[...]
    outs = [x + pre[:, i * A:(i + 1) * A] for i in range(K)]
    outs_bf = [v.astype(jnp.bfloat16).astype(jnp.float32) for v in outs]
    gated = []
    for j in range(2 * K):
        gj = None
        for i in range(K):
            w = jnp.full((1, 1), wfc_ref[i, j], jnp.float32).astype(
                jnp.bfloat16).astype(jnp.float32)
            gj = outs_bf[i] * w if gj is None else gj + outs_bf[i] * w
        gated.append(gj + bfc_ref[j])
    mg = jnp.maximum(jnp.maximum(gated[K], gated[K + 1]), gated[K + 2])
    es = [jnp.exp(gated[K + t] - mg) for t in range(K)]
    se = es[0] + es[1] + es[2]
    xn_ref[...] = (gated[0] * es[0] + gated[1] * es[1]
                   + gated[2] * es[2]) / se


def _atom(x, pre, c2, s2sum, s2sq, g2g, g2b, wfc, bfc):
    return pl.pallas_call(
        _atom_body,
        grid=(GRID,),
        in_specs=[
            pl.BlockSpec((TA, A), lambda i: (i, 0)),
            pl.BlockSpec((TA, KA), lambda i: (i, 0)),
            pl.BlockSpec((1, KA), lambda i: (0, 0)),
            pl.BlockSpec((1, KA), lambda i: (0, 0)),
            pl.BlockSpec((1, KA), lambda i: (0, 0)),
            pl.BlockSpec((1, KA), lambda i: (0, 0)),
            pl.BlockSpec((1, KA), lambda i: (0, 0)),
            pl.BlockSpec(memory_space=pltpu.SMEM),
            pl.BlockSpec(memory_space=pltpu.SMEM),
        ],
        out_specs=pl.BlockSpec((TA, A), lambda i: (i, 0)),
        out_shape=jax.ShapeDtypeStruct((N, A), jnp.float32),
        compiler_params=pltpu.CompilerParams(
            dimension_semantics=("arbitrary",)
        ),
    )(x, pre, c2, s2sum, s2sq, g2g, g2b, wfc, bfc)


def _pool_body(x_ref, oh_ref, w1_ref, b1_ref, w2_ref, b2_ref, w3_ref, b3_ref,
               wo_ref, bo_ref, out_ref, acc, cacc):
    oh = oh_ref[...]
    x = x_ref[...]
    dn = (((0,), (0,)), ((), ()))
    ps = lax.dot_general(oh, x, dn, preferred_element_type=jnp.float32,
                         precision=lax.Precision.HIGHEST)
    pc = lax.dot_general(oh, jnp.ones_like(x), dn,
                         preferred_element_type=jnp.float32,
                         precision=lax.Precision.HIGHEST)
    i = pl.program_id(0)

    @pl.when(i == 0)
    def _():
        acc[...] = ps
        cacc[...] = pc

    @pl.when(i > 0)
    def _():
        acc[...] = acc[...] + ps
        cacc[...] = cacc[...] + pc

    @pl.when(i == GRID - 1)
    def _():
        crys = acc[...] / jnp.maximum(cacc[...], 1.0)
        crys = jnp.maximum(crys, 0.0)
        crys = jnp.dot(crys, w1_ref[...],
                       preferred_element_type=jnp.float32) + b1_ref[...]
        crys = jnp.maximum(crys, 0.0)
        crys = jnp.dot(crys, w2_ref[...],
                       preferred_element_type=jnp.float32) + b2_ref[...]
        crys = jnp.maximum(crys, 0.0)
        crys = jnp.dot(crys, w3_ref[...],
                       preferred_element_type=jnp.float32) + b3_ref[...]
        crys = jnp.maximum(crys, 0.0)
        crys_bf = crys.astype(jnp.bfloat16).astype(jnp.float32)
        wo_bf = wo_ref[...].astype(jnp.bfloat16).astype(jnp.float32)
        out_ref[...] = jnp.sum(crys_bf * wo_bf, axis=1,
                               keepdims=True) + bo_ref[0]


def _pool(x, onehot, w1, b1, w2, b2, w3, b3, woT, bo):
    dim2 = w1.shape[1]
    return pl.pallas_call(
        _pool_body,
        grid=(GRID,),
        in_specs=[
            pl.BlockSpec((TA, A), lambda i: (i, 0)),
            pl.BlockSpec((TA, NCRYS), lambda i: (i, 0)),
            pl.BlockSpec((A, dim2), lambda i: (0, 0)),
            pl.BlockSpec((1, dim2), lambda i: (0, 0)),
            pl.BlockSpec((dim2, dim2), lambda i: (0, 0)),
            pl.BlockSpec((1, dim2), lambda i: (0, 0)),
            pl.BlockSpec((dim2, dim2), lambda i: (0, 0)),
            pl.BlockSpec((1, dim2), lambda i: (0, 0)),
            pl.BlockSpec((1, dim2), lambda i: (0, 0)),
            pl.BlockSpec(memory_space=pltpu.SMEM),
        ],
        out_specs=pl.BlockSpec((NCRYS, 1), lambda i: (0, 0)),
        out_shape=jax.ShapeDtypeStruct((NCRYS, 1), jnp.float32),
        scratch_shapes=[
            pltpu.VMEM((NCRYS, A), jnp.float32),
            pltpu.VMEM((NCRYS, A), jnp.float32),
        ],
        compiler_params=pltpu.CompilerParams(
            dimension_semantics=("arbitrary",)
        ),
    )(x, onehot, w1, b1, w2, b2, w3, b3, woT, bo)


def _forward_impl(atom_fea, nbr_fea, nbr_fea_idx, batch, params, gather_fn):
    x, bond = _prep(atom_fea, nbr_fea, params["emb_W"],
                    params["emb_b"].reshape(1, A))
    idx_flat = nbr_fea_idx.reshape(-1).astype(jnp.int32)
    idx_pad = jnp.concatenate(
        [idx_flat, jnp.zeros((_BPAD - N * M,), jnp.int32)])
    onehot = (batch[:, None] == jnp.arange(NCRYS, dtype=batch.dtype)[None, :]
              ).astype(jnp.float32)

    for p in params["convs"]:
        w_all = jnp.concatenate([p["fc_W"][i] for i in range(K)], axis=1)
        bias = jnp.concatenate(p["fc_b"]).reshape(1, KD)
        g1g = jnp.concatenate(p["bn1_g"]).reshape(1, KD)
        g1b = jnp.concatenate(p["bn1_b"]).reshape(1, KD)
        g2g = jnp.concatenate(p["bn2_g"]).reshape(1, KA)
        g2b = jnp.concatenate(p["bn2_b"]).reshape(1, KA)

        wg = jnp.kron(p["nbr_fc_W"], jnp.eye(Bf, dtype=jnp.float32))
        bg = jnp.repeat(p["nbr_fc_b"], Bf).reshape(1, 2 * K * Bf)

        gath = gather_fn(x, idx_pad)[:N * M]
        c, ssum, ssq, g_all = _stats(x, gath, bond, w_all, bias)
        pre, bond, c2, s2sum, s2sq = _main(g_all, bond,
                                           c, ssum, ssq, g1g, g1b, wg, bg)
        x = _atom(x, pre, c2, s2sum, s2sq, g2g, g2b,
                  p["atom_fc_W"], p["atom_fc_b"])

    return _pool(x, onehot,
                 params["c2f_W"], params["c2f_b"].reshape(1, -1),
                 params["fcs_W"][0], params["fcs_b"][0].reshape(1, -1),
                 params["fcs_W"][1], params["fcs_b"][1].reshape(1, -1),
                 params["out_W"].reshape(1, -1), params["out_b"])


@jax.jit
def kernel(atom_fea, nbr_fea, nbr_fea_idx, batch, params):
    return _forward_impl(atom_fea, nbr_fea, nbr_fea_idx, batch, params,
                         _sc_gather)
